# Initial kernel scaffold; baseline (speedup 1.0000x reference)
import jax
import jax.numpy as jnp
from jax.experimental import pallas as pl


def kernel(x, edge_index, edge_weight, conv1_W, conv1_b, bn1_g, bn1_b, gat_W, gat_asrc, gat_adst, gat_b, bn2_g, bn2_b, conv3_W, conv3_b, bn3_g, bn3_b, conv4_W, conv4_b, proj_W, proj_b, gate_W1, gate_b1, gate_W2, gate_b2, sp_W, sp_b):
    raise NotImplementedError("write your pallas kernel here")


# speedup vs baseline: 18.3572x; 18.3572x over previous
import functools

import jax
import jax.numpy as jnp
from jax import lax
from jax.experimental import pallas as pl
from jax.experimental.pallas import tpu as pltpu
from jax.experimental.pallas import tpu_sc as plsc

N = 50000
E = 800000
EP = E + N
NC, NS, L = 2, 16, 16
NW = NC * NS
NR = 64
RNG = 784
NPAD = NR * RNG
B = 128
BG = 64
EPC = 851968
VETO = 2.5

_SC_PARAMS = pltpu.CompilerParams(needs_layout_passes=False,
                                  use_tc_tiling_on_sc=False)


def _mesh():
    return plsc.VectorSubcoreMesh(core_axis_name="c", subcore_axis_name="s",
                                  num_cores=NC, num_subcores=NS)


def _wid():
    return lax.axis_index("s") * NC + lax.axis_index("c")


def _iota():
    return lax.iota(jnp.int32, L)


def _onehot(k):
    return (lax.iota(jnp.int32, L) == k).astype(jnp.float32)


def _range_bounds(es_v, r):
    ev = es_v[pl.ds(r, L)]
    e0 = ev[0]
    e1 = ev[1]
    e0a = (e0 // 8) * 8
    return e0, e1, e0a


def _sc_deg(dsts, ews, es):
    @functools.partial(
        pl.kernel,
        out_type=jax.ShapeDtypeStruct((NPAD, L), jnp.float32),
        mesh=_mesh(),
        scratch_types=[
            pltpu.VMEM((80,), jnp.int32),
            pltpu.VMEM((B,), jnp.int32),
            pltpu.VMEM((B,), jnp.float32),
            pltpu.VMEM((RNG, L), jnp.float32),
            pltpu.SemaphoreType.DMA,
            pltpu.SemaphoreType.DMA,
        ],
        compiler_params=_SC_PARAMS,
    )
    def k(dst_h, ew_h, es_h, deg_h, es_v, dst_v, w_v, acc, sem, semb):
        w = _wid()
        pltpu.sync_copy(es_h, es_v.at[pl.ds(0, 72)])
        e0c = _onehot(0)
        for rr in range(2):
            r = w * 2 + rr
            base = r * RNG
            e0, e1, e0a = _range_bounds(es_v, r)
            nb = (e1 - e0a + B - 1) // B

            def zb(i, _):
                acc[i, pl.ds(0, L)] = jnp.zeros((L,), jnp.float32)
                return 0
            lax.fori_loop(0, RNG, zb, 0)

            def bb(i, _):
                eb = e0a + i * B
                ca = pltpu.async_copy(dst_h.at[pl.ds(eb, B)], dst_v, sem)
                cb = pltpu.async_copy(ew_h.at[pl.ds(eb, B)], w_v, semb)
                ca.wait()
                cb.wait()

                def gb(g, _):
                    ids = _iota() + (eb + g * L)
                    dvec = dst_v[pl.ds(g * L, L)]
                    wvec = w_v[pl.ds(g * L, L)]
                    valid = (ids >= e0) & (ids < e1)
                    wvec = jnp.where(valid, wvec, 0.0)
                    dloc = jnp.clip(dvec - base, 0, RNG - 1)
                    for u in range(L):
                        plsc.addupdate(acc.at[dloc[u]], wvec[u] * e0c)
                    return 0
                lax.fori_loop(0, B // L, gb, 0)
                return 0
            lax.fori_loop(0, nb, bb, 0)
            pltpu.sync_copy(acc, deg_h.at[pl.ds(base, RNG)])

    return k(dsts, ews, es)


def _sc_norm(srcs, dsts, ews, dinv):
    NB_STATIC = EPC // NW // B

    @functools.partial(
        pl.kernel,
        out_type=jax.ShapeDtypeStruct((EPC,), jnp.float32),
        mesh=_mesh(),
        scratch_types=[
            pltpu.VMEM((NPAD,), jnp.float32),
            pltpu.VMEM((B,), jnp.int32),
            pltpu.VMEM((B,), jnp.int32),
            pltpu.VMEM((B,), jnp.float32),
            pltpu.VMEM((B,), jnp.float32),
            pltpu.SemaphoreType.DMA,
            pltpu.SemaphoreType.DMA,
            pltpu.SemaphoreType.DMA,
        ],
        compiler_params=_SC_PARAMS,
    )
    def k(src_h, dst_h, ew_h, dinv_h, norm_h,
          dinv_v, src_v, dst_v, w_v, out_v, sem, semb, semc):
        w = _wid()
        pltpu.sync_copy(dinv_h, dinv_v)
        base_e = w * (EPC // NW)

        def bb(i, _):
            eb = base_e + i * B
            ca = pltpu.async_copy(src_h.at[pl.ds(eb, B)], src_v, sem)
            cb = pltpu.async_copy(dst_h.at[pl.ds(eb, B)], dst_v, semb)
            cc = pltpu.async_copy(ew_h.at[pl.ds(eb, B)], w_v, semc)
            ca.wait()
            cb.wait()
            cc.wait()

            def gb(g, _):
                s16 = src_v[pl.ds(g * L, L)]
                d16 = dst_v[pl.ds(g * L, L)]
                w16 = w_v[pl.ds(g * L, L)]
                n16 = (plsc.load_gather(dinv_v, [s16]) * w16 *
                       plsc.load_gather(dinv_v, [d16]))
                out_v[pl.ds(g * L, L)] = n16
                return 0
            lax.fori_loop(0, B // L, gb, 0)
            pltpu.sync_copy(out_v, norm_h.at[pl.ds(eb, B)])
            return 0
        lax.fori_loop(0, NB_STATIC, bb, 0)

    return k(srcs, dsts, ews, dinv)


def _sc_agg(h, srcs, dsts, wts, es, width, init=None):
    nblk = width // L
    n_in = 6 if init is not None else 5

    @functools.partial(
        pl.kernel,
        out_type=jax.ShapeDtypeStruct((NPAD, width), jnp.float32),
        mesh=_mesh(),
        scratch_types=[
            pltpu.VMEM((80,), jnp.int32),
            pltpu.VMEM((B,), jnp.int32),
            pltpu.VMEM((B,), jnp.int32),
            pltpu.VMEM((B,), jnp.float32),
            pltpu.VMEM((B, width), jnp.float32),
            pltpu.VMEM((RNG, width), jnp.float32),
            pltpu.SemaphoreType.DMA,
            pltpu.SemaphoreType.DMA,
            pltpu.SemaphoreType.DMA,
            pltpu.SemaphoreType.DMA,
        ],
        compiler_params=_SC_PARAMS,
    )
    def k(*refs):
        if init is not None:
            (h_h, src_h, dst_h, w_h, es_h, init_h, out_h,
             es_v, idx_v, dst_v, w_v, rows_v, acc, sem, semb, semc,
             semg) = refs
        else:
            (h_h, src_h, dst_h, w_h, es_h, out_h,
             es_v, idx_v, dst_v, w_v, rows_v, acc, sem, semb, semc,
             semg) = refs
        w = _wid()
        pltpu.sync_copy(es_h, es_v.at[pl.ds(0, 72)])
        for rr in range(2):
            r = w * 2 + rr
            base = r * RNG
            e0, e1, e0a = _range_bounds(es_v, r)
            nb = (e1 - e0a + B - 1) // B

            if init is not None:
                pltpu.sync_copy(init_h.at[pl.ds(base, RNG)], acc)
            else:
                def zb(i, _):
                    for p in range(nblk):
                        acc[i, pl.ds(p * L, L)] = jnp.zeros((L,), jnp.float32)
                    return 0
                lax.fori_loop(0, RNG, zb, 0)

            def bb(i, _):
                eb = e0a + i * B
                ca = pltpu.async_copy(src_h.at[pl.ds(eb, B)], idx_v, sem)
                cb = pltpu.async_copy(dst_h.at[pl.ds(eb, B)], dst_v, semb)
                cc = pltpu.async_copy(w_h.at[pl.ds(eb, B)], w_v, semc)
                ca.wait()
                cd = pltpu.async_copy(h_h.at[idx_v], rows_v, semg)
                cb.wait()
                cc.wait()
                cd.wait()

                def gb(g, _):
                    ids = _iota() + (eb + g * L)
                    dvec = dst_v[pl.ds(g * L, L)]
                    wvec = w_v[pl.ds(g * L, L)]
                    valid = (ids >= e0) & (ids < e1)
                    wvec = jnp.where(valid, wvec, 0.0)
                    dloc = jnp.clip(dvec - base, 0, RNG - 1)
                    for u in range(L):
                        j = g * L + u
                        wu = wvec[u]
                        du = dloc[u]
                        for p in range(nblk):
                            val = rows_v[j, pl.ds(p * L, L)] * wu
                            plsc.addupdate(acc.at[du, pl.ds(p * L, L)], val)
                    return 0
                lax.fori_loop(0, B // L, gb, 0)
                return 0
            lax.fori_loop(0, nb, bb, 0)
            pltpu.sync_copy(acc, out_h.at[pl.ds(base, RNG)])

    args = (h, srcs, dsts, wts, es) + ((init,) if init is not None else ())
    del n_in
    return k(*args)


def _sc_gat_prep(srcs, dsts, asd, es):
    @functools.partial(
        pl.kernel,
        out_type=(jax.ShapeDtypeStruct((EPC, 4), jnp.float32),
                  jax.ShapeDtypeStruct((NR, L), jnp.float32)),
        mesh=_mesh(),
        scratch_types=[
            pltpu.VMEM((80,), jnp.int32),
            pltpu.VMEM((B,), jnp.int32),
            pltpu.VMEM((B,), jnp.int32),
            pltpu.VMEM((B, L), jnp.float32),
            pltpu.VMEM((B, L), jnp.float32),
            pltpu.VMEM((B, 4), jnp.float32),
            pltpu.VMEM((2, L), jnp.float32),
            pltpu.SemaphoreType.DMA,
            pltpu.SemaphoreType.DMA,
            pltpu.SemaphoreType.DMA,
            pltpu.SemaphoreType.DMA,
        ],
        compiler_params=_SC_PARAMS,
    )
    def k(src_h, dst_h, asd_h, es_h, e4_h, maxs_h,
          es_v, src_v, dst_v, as_rows, ad_rows, e_st, mrow_st, sem, semb,
          semg, semg2):
        w = _wid()
        pltpu.sync_copy(es_h, es_v.at[pl.ds(0, 72)])
        eids0 = _iota()
        for rr in range(2):
            r = w * 2 + rr
            e0, e1, e0a = _range_bounds(es_v, r)
            nb = (e1 - e0a + B - 1) // B
            neg = jnp.full((L,), -1e30, jnp.float32)

            def bb(i, carry):
                m0, m1, m2, m3 = carry
                eb = e0a + i * B
                ca = pltpu.async_copy(src_h.at[pl.ds(eb, B)], src_v, sem)
                cb = pltpu.async_copy(dst_h.at[pl.ds(eb, B)], dst_v, semb)
                ca.wait()
                cb.wait()
                cc = pltpu.async_copy(asd_h.at[src_v], as_rows, semg)
                cd = pltpu.async_copy(asd_h.at[dst_v], ad_rows, semg2)
                cc.wait()
                cd.wait()

                def gb(g, mc):
                    eids = eids0 + g * L
                    ids = eids + eb
                    valid = (ids >= e0) & (ids < e1)
                    ms = []
                    for hh in range(4):
                        c1 = jnp.full((L,), hh, jnp.int32)
                        c2 = jnp.full((L,), 4 + hh, jnp.int32)
                        a1 = plsc.load_gather(as_rows, [eids, c1])
                        a2 = plsc.load_gather(ad_rows, [eids, c2])
                        s = a1 + a2
                        e16 = jnp.where(s > 0, s, 0.2 * s)
                        plsc.store_scatter(e_st, [eids, c1], e16)
                        ms.append(jnp.maximum(mc[hh],
                                              jnp.where(valid, e16, neg)))
                    return tuple(ms)
                m0, m1, m2, m3 = lax.fori_loop(0, B // L, gb,
                                               (m0, m1, m2, m3))
                pltpu.sync_copy(e_st, e4_h.at[pl.ds(eb, B)])
                return m0, m1, m2, m3

            mf = lax.fori_loop(0, nb, bb, (neg, neg, neg, neg))
            mrow = jnp.zeros((L,), jnp.float32)
            for hh in range(4):
                mrow = mrow + jnp.max(mf[hh]) * _onehot(hh)
            mrow_st[rr, pl.ds(0, L)] = mrow
            pltpu.sync_copy(mrow_st.at[rr], maxs_h.at[r])

    return k(srcs, dsts, asd, es)


def _sc_gat_agg(h2, srcs, dsts, e4, es, maxs):
    @functools.partial(
        pl.kernel,
        out_type=jax.ShapeDtypeStruct((NPAD, 128), jnp.float32),
        mesh=_mesh(),
        scratch_types=[
            pltpu.VMEM((80,), jnp.int32),
            pltpu.VMEM((BG,), jnp.int32),
            pltpu.VMEM((BG,), jnp.int32),
            pltpu.VMEM((BG, 4), jnp.float32),
            pltpu.VMEM((BG, 128), jnp.float32),
            pltpu.VMEM((RNG, 128), jnp.float32),
            pltpu.VMEM((RNG, L), jnp.float32),
            pltpu.VMEM((L,), jnp.float32),
            pltpu.SemaphoreType.DMA,
            pltpu.SemaphoreType.DMA,
            pltpu.SemaphoreType.DMA,
            pltpu.SemaphoreType.DMA,
        ],
        compiler_params=_SC_PARAMS,
    )
    def k(h_h, src_h, dst_h, e4_h, es_h, maxs_h, out_h,
          es_v, idx_v, dst_v, e_v, rows_v, acc, den, mrow_v, sem, semb,
          semc, semg):
        w = _wid()
        pltpu.sync_copy(es_h, es_v.at[pl.ds(0, 72)])
        eids0 = _iota()
        ohs = [_onehot(hh) for hh in range(4)]
        for rr in range(2):
            r = w * 2 + rr
            base = r * RNG
            e0, e1, e0a = _range_bounds(es_v, r)
            nb = (e1 - e0a + BG - 1) // BG
            pltpu.sync_copy(maxs_h.at[r], mrow_v)
            mrow = mrow_v[pl.ds(0, L)]

            def zb(i, _):
                for p in range(8):
                    acc[i, pl.ds(p * L, L)] = jnp.zeros((L,), jnp.float32)
                den[i, pl.ds(0, L)] = jnp.zeros((L,), jnp.float32)
                return 0
            lax.fori_loop(0, RNG, zb, 0)

            def bb(i, _):
                eb = e0a + i * BG
                ca = pltpu.async_copy(src_h.at[pl.ds(eb, BG)], idx_v, sem)
                cb = pltpu.async_copy(dst_h.at[pl.ds(eb, BG)], dst_v, semb)
                cc = pltpu.async_copy(e4_h.at[pl.ds(eb, BG)], e_v, semc)
                ca.wait()
                cd = pltpu.async_copy(h_h.at[idx_v], rows_v, semg)
                cb.wait()
                cc.wait()
                cd.wait()

                def gb(g, _):
                    eids = eids0 + g * L
                    ids = eids + eb
                    valid = (ids >= e0) & (ids < e1)
                    dvec = dst_v[pl.ds(g * L, L)]
                    dloc = jnp.clip(dvec - base, 0, RNG - 1)
                    exs = []
                    for hh in range(4):
                        ch = jnp.full((L,), hh, jnp.int32)
                        ecol = plsc.load_gather(e_v, [eids, ch])
                        ex16 = jnp.exp(ecol - mrow[hh])
                        exs.append(jnp.where(valid, ex16, 0.0))
                    for u in range(L):
                        j = g * L + u
                        du = dloc[u]
                        dv = jnp.zeros((L,), jnp.float32)
                        for hh in range(4):
                            exu = exs[hh][u]
                            for p in (2 * hh, 2 * hh + 1):
                                val = rows_v[j, pl.ds(p * L, L)] * exu
                                plsc.addupdate(acc.at[du, pl.ds(p * L, L)],
                                               val)
                            dv = dv + exu * ohs[hh]
                        plsc.addupdate(den.at[du], dv)
                    return 0
                lax.fori_loop(0, BG // L, gb, 0)
                return 0
            lax.fori_loop(0, nb, bb, 0)

            def db(i, _):
                drow = den[i, pl.ds(0, L)]
                rv = 1.0 / (drow + 1e-16)
                for hh in range(4):
                    rvh = rv[hh]
                    for p in (2 * hh, 2 * hh + 1):
                        acc[i, pl.ds(p * L, L)] = (
                            acc[i, pl.ds(p * L, L)] * rvh)
                return 0
            lax.fori_loop(0, RNG, db, 0)
            pltpu.sync_copy(acc, out_h.at[pl.ds(base, RNG)])

    return k(h2, srcs, dsts, e4, es, maxs)


_BR = 1024
_GRID = NPAD // _BR


def _row_spec(width):
    return pl.BlockSpec((_BR, width), lambda i: (i, 0))


def _full_spec(shape):
    nd = len(shape)
    return pl.BlockSpec(shape, lambda i: (0,) * nd)


def _tc1(xp, w1p, projp, proj_b, deg):
    def body(x_ref, w1_ref, pj_ref, pb_ref, dg_ref, h1_ref, xp_ref, di_ref):
        xb = x_ref[...]
        h1_ref[...] = jnp.dot(xb, w1_ref[...],
                              preferred_element_type=jnp.float32,
                              precision=lax.Precision.HIGHEST)
        xp_ref[...] = jnp.dot(xb, pj_ref[...],
                              preferred_element_type=jnp.float32,
                              precision=lax.Precision.HIGHEST) + pb_ref[...]
        dg = dg_ref[...][:, :1]
        di_ref[...] = jnp.where(dg > 0, lax.rsqrt(jnp.maximum(dg, 1e-30)),
                                0.0)

    return pl.pallas_call(
        body,
        grid=(_GRID,),
        in_specs=[_row_spec(32), _full_spec((32, 128)), _full_spec((32, 128)),
                  _full_spec((128,)), _row_spec(L)],
        out_specs=[_row_spec(128), _row_spec(128), _row_spec(1)],
        out_shape=[jax.ShapeDtypeStruct((NPAD, 128), jnp.float32),
                   jax.ShapeDtypeStruct((NPAD, 128), jnp.float32),
                   jax.ShapeDtypeStruct((NPAD, 1), jnp.float32)],
    )(xp, w1p, projp, proj_b, deg)


def _tc_stats(t):
    def body(t_ref, o_ref):
        @pl.when(pl.program_id(0) == 0)
        def _():
            o_ref[...] = jnp.zeros_like(o_ref)
        tb = t_ref[...]
        s = jnp.sum(tb, axis=0, keepdims=True)
        sq = jnp.sum(tb * tb, axis=0, keepdims=True)
        o_ref[...] += jnp.concatenate(
            [s, sq, jnp.zeros((6, 128), jnp.float32)], axis=0)

    return pl.pallas_call(
        body,
        grid=(_GRID,),
        in_specs=[_row_spec(128)],
        out_specs=pl.BlockSpec((8, 128), lambda i: (0, 0)),
        out_shape=jax.ShapeDtypeStruct((8, 128), jnp.float32),
    )(t)


def _bn_from_stats(agg, st, badd, g, bb):
    mean = st[0] / N + badd
    var = st[1] / N + badd * badd + 2 * badd * (st[0] / N) - mean * mean
    inv = lax.rsqrt(var + 1e-5) * g
    return (agg + badd - mean) * inv + bb


def _tc3(agg1, st1, b1, g1, bb1, gatw, a_s, a_d):
    def body(a_ref, st_ref, b_ref, g_ref, bb_ref, w_ref, as_ref, ad_ref,
             x1_ref, h2_ref, asd_ref):
        st = st_ref[...]
        x1 = jax.nn.relu(_bn_from_stats(a_ref[...], st, b_ref[...],
                                        g_ref[...], bb_ref[...]))
        x1_ref[...] = x1
        h2 = jnp.dot(x1, w_ref[...], preferred_element_type=jnp.float32,
                     precision=lax.Precision.HIGHEST)
        h2_ref[...] = h2
        asb = jnp.dot(h2, as_ref[...], preferred_element_type=jnp.float32,
                      precision=lax.Precision.HIGHEST)
        adb = jnp.dot(h2, ad_ref[...], preferred_element_type=jnp.float32,
                      precision=lax.Precision.HIGHEST)
        asd_ref[...] = jnp.concatenate(
            [asb, adb, jnp.zeros((_BR, 8), jnp.float32)], axis=1)

    return pl.pallas_call(
        body,
        grid=(_GRID,),
        in_specs=[_row_spec(128), _full_spec((8, 128)), _full_spec((128,)),
                  _full_spec((128,)), _full_spec((128,)),
                  _full_spec((128, 128)), _full_spec((128, 4)),
                  _full_spec((128, 4))],
        out_specs=[_row_spec(128), _row_spec(128), _row_spec(L)],
        out_shape=[jax.ShapeDtypeStruct((NPAD, 128), jnp.float32),
                   jax.ShapeDtypeStruct((NPAD, 128), jnp.float32),
                   jax.ShapeDtypeStruct((NPAD, L), jnp.float32)],
    )(agg1, st1, b1, g1, bb1, gatw, a_s, a_d)


def _tc4(aggGd, st2, gat_b, g2, bb2, xproj, w3):
    def body(a_ref, st_ref, b_ref, g_ref, bb_ref, xp_ref, w_ref, h3_ref):
        st = st_ref[...]
        x2 = jax.nn.relu(_bn_from_stats(a_ref[...], st, b_ref[...],
                                        g_ref[...], bb_ref[...]))
        x2 = x2 + xp_ref[...]
        h3_ref[...] = jnp.dot(x2, w_ref[...],
                              preferred_element_type=jnp.float32,
                              precision=lax.Precision.HIGHEST)

    return pl.pallas_call(
        body,
        grid=(_GRID,),
        in_specs=[_row_spec(128), _full_spec((8, 128)), _full_spec((128,)),
                  _full_spec((128,)), _full_spec((128,)), _row_spec(128),
                  _full_spec((128, 128))],
        out_specs=_row_spec(128),
        out_shape=jax.ShapeDtypeStruct((NPAD, 128), jnp.float32),
    )(aggGd, st2, gat_b, g2, bb2, xproj, w3)


def _tc5(agg3, st3, b3, g3, bb3, x1, xp, w4p, gw1, gb1, gw2p, gb2p,
         spw_row, spb_row, b4p):
    def body(a_ref, st_ref, b_ref, g_ref, bb_ref, x1_ref, x_ref, w4_ref,
             g1_ref, gb1_ref, g2_ref, gb2_ref, sw_ref, sb_ref, b4_ref,
             h4_ref, ex_ref):
        st = st_ref[...]
        x3 = jax.nn.relu(_bn_from_stats(a_ref[...], st, b_ref[...],
                                        g_ref[...], bb_ref[...]))
        x3 = x3 + x1_ref[...]
        h4_ref[...] = jnp.dot(x3, w4_ref[...],
                              preferred_element_type=jnp.float32,
                              precision=lax.Precision.HIGHEST)
        z1 = jax.nn.relu(jnp.dot(x3, g1_ref[...],
                                 preferred_element_type=jnp.float32,
                                 precision=lax.Precision.HIGHEST)
                         + gb1_ref[...])
        z2 = jnp.dot(z1, g2_ref[...], preferred_element_type=jnp.float32,
                     precision=lax.Precision.HIGHEST) + gb2_ref[...]
        gate = 1.0 / (1.0 + jnp.exp(-z2))
        status = x_ref[...][:, 12:13]
        flag = (status < 0.5).astype(jnp.float32)
        sev = 1.0 - status
        sig = sev * sw_ref[...] + sb_ref[...]
        ex_ref[...] = VETO * flag * gate * sig + b4_ref[...]

    return pl.pallas_call(
        body,
        grid=(_GRID,),
        in_specs=[_row_spec(128), _full_spec((8, 128)), _full_spec((128,)),
                  _full_spec((128,)), _full_spec((128,)), _row_spec(128),
                  _row_spec(32), _full_spec((128, L)),
                  _full_spec((128, 64)), _full_spec((64,)),
                  _full_spec((64, L)), _full_spec((L,)),
                  _full_spec((L,)), _full_spec((L,)), _full_spec((L,))],
        out_specs=[_row_spec(L), _row_spec(L)],
        out_shape=[jax.ShapeDtypeStruct((NPAD, L), jnp.float32),
                   jax.ShapeDtypeStruct((NPAD, L), jnp.float32)],
    )(agg3, st3, b3, g3, bb3, x1, xp, w4p, gw1, gb1, gw2p, gb2p,
      spw_row, spb_row, b4p)


def kernel(x, edge_index, edge_weight, conv1_W, conv1_b, bn1_g, bn1_b,
           gat_W, gat_asrc, gat_adst, gat_b, bn2_g, bn2_b, conv3_W, conv3_b,
           bn3_g, bn3_b, conv4_W, conv4_b, proj_W, proj_b, gate_W1, gate_b1,
           gate_W2, gate_b2, sp_W, sp_b):
    f32 = jnp.float32
    loop = jnp.arange(N, dtype=edge_index.dtype)
    src_all = jnp.concatenate([edge_index[0], loop])
    dst_all = jnp.concatenate([edge_index[1], loop])
    ew_all = jnp.concatenate([edge_weight, jnp.ones((N,), f32)])
    perm = jnp.argsort(dst_all)
    srcs = jnp.concatenate(
        [src_all[perm], jnp.full((EPC - EP,), NPAD - 1, jnp.int32)])
    dsts_r = dst_all[perm]
    dsts = jnp.concatenate(
        [dsts_r, jnp.full((EPC - EP,), NPAD - 1, jnp.int32)])
    ews = jnp.concatenate([ew_all[perm], jnp.zeros((EPC - EP,), f32)])
    bounds = jnp.arange(0, NPAD, RNG, dtype=jnp.int32)
    es = jnp.searchsorted(dsts_r, bounds).astype(jnp.int32)
    es = jnp.concatenate(
        [es, jnp.array([EP], jnp.int32), jnp.zeros((72 - NR - 1,),
                                                   jnp.int32)])

    xp = jnp.pad(x, ((0, NPAD - N), (0, 32 - x.shape[1])))
    w1p = jnp.pad(conv1_W, ((0, 32 - conv1_W.shape[0]), (0, 0)))
    projp = jnp.pad(proj_W, ((0, 32 - proj_W.shape[0]), (0, 0)))

    deg = _sc_deg(dsts, ews, es)
    h1, xproj, dinv = _tc1(xp, w1p, projp, proj_b, deg)
    norm = _sc_norm(srcs, dsts, ews, dinv[:, 0])
    agg1 = _sc_agg(h1, srcs, dsts, norm, es, 128)
    st1 = _tc_stats(agg1)
    a_s = (gat_asrc[:, :, None] * jnp.eye(4, dtype=f32)[:, None, :]
           ).reshape(128, 4)
    a_d = (gat_adst[:, :, None] * jnp.eye(4, dtype=f32)[:, None, :]
           ).reshape(128, 4)
    x1, h2, asd = _tc3(agg1, st1, conv1_b, bn1_g, bn1_b, gat_W, a_s, a_d)
    e4, maxs = _sc_gat_prep(srcs, dsts, asd, es)
    aggGd = _sc_gat_agg(h2, srcs, dsts, e4, es, maxs)
    st2 = _tc_stats(aggGd)
    h3 = _tc4(aggGd, st2, gat_b, bn2_g, bn2_b, xproj, conv3_W)
    agg3 = _sc_agg(h3, srcs, dsts, norm, es, 128)
    st3 = _tc_stats(agg3)
    w4p = jnp.pad(conv4_W, ((0, 0), (0, L - conv4_W.shape[1])))
    gw2p = jnp.pad(gate_W2, ((0, 0), (0, L - gate_W2.shape[1])))
    gb2p = jnp.pad(gate_b2, (0, L - gate_b2.shape[0]))
    spw_row = jnp.pad(sp_W[0], (0, L - sp_W.shape[1]))
    spb_row = jnp.pad(sp_b, (0, L - sp_b.shape[0]))
    b4p = jnp.pad(conv4_b, (0, L - conv4_b.shape[0]))
    h4p, extra = _tc5(agg3, st3, conv3_b, bn3_g, bn3_b, x1, xp, w4p,
                      gate_W1, gate_b1, gw2p, gb2p, spw_row, spb_row, b4p)
    out16 = _sc_agg(h4p, srcs, dsts, norm, es, L, init=extra)
    return out16[:N, :12]
